# baseline (speedup 1.0000x reference)
import functools

import jax
import jax.numpy as jnp
from jax import lax
from jax.experimental import pallas as pl
from jax.experimental.pallas import tpu as pltpu
from jax.experimental.pallas import tpu_sc as plsc

NUM_CODES = 8192
DIM = 256
N_TOKENS = 8192
COMMIT = 0.25


_BN = 2048
_BK = 1024
_KB = NUM_CODES // _BK
_BK2 = 2048
_KE = NUM_CODES // _BK2


_NB = N_TOKENS // _BN


def _fused_body(x_ref, e_ref, idx_ref, enc_ref, loss_ref,
                minv_ref, colf_ref, coli_ref, idxp_ref, xn_ref):
    i = pl.program_id(0)
    j = pl.program_id(1)

    @pl.when(jnp.logical_and(i == 0, j == 0))
    def _():
        it = lax.broadcasted_iota(jnp.int32, (8, _BK), 1)
        coli_ref[...] = it
        colf_ref[...] = it.astype(jnp.float32)

    @pl.when(i > 0)
    def _():
        idx_adj = idxp_ref[...] - j * _BK
        eq = coli_ref[0:1, :] == idx_adj
        enc_ref[...] = jnp.where(eq, jnp.float32(1.0), jnp.float32(0.0))

    x = x_ref[...]
    e = e_ref[...]
    en = jnp.sum(e * e, axis=1)
    e2 = -2.0 * e
    mm2 = lax.dot_general(x, e2, (((1,), (1,)), ((), ())),
                          preferred_element_type=jnp.float32)

    @pl.when(j == 0)
    def _():
        xn_ref[...] = jnp.sum(x * x, axis=1, keepdims=True)

    dist = (xn_ref[...] + en[None, :]) + mm2
    m = jnp.min(dist, axis=1, keepdims=True)
    rowf = jnp.min(
        jnp.where(dist == m, colf_ref[0:1, :], jnp.float32(_BK)),
        axis=1, keepdims=True)
    lidx = rowf.astype(jnp.int32) + j * _BK

    @pl.when(j == 0)
    def _():
        minv_ref[...] = m
        idx_ref[...] = lidx

    @pl.when(j != 0)
    def _():
        better = m < minv_ref[...]
        idx_ref[...] = jnp.where(better, lidx, idx_ref[...])
        minv_ref[...] = jnp.where(better, m, minv_ref[...])

    @pl.when(j == _KB - 1)
    def _():
        idxp_ref[...] = idx_ref[...]
        s = jnp.sum(minv_ref[...])

        @pl.when(i == 0)
        def _():
            loss_ref[0, 0] = s

        @pl.when(i != 0)
        def _():
            loss_ref[0, 0] += s


def _run_fused(flat_x, emb):
    return pl.pallas_call(
        _fused_body,
        grid=(_NB, _KB),
        in_specs=[
            pl.BlockSpec((_BN, DIM), lambda i, j: (i, 0)),
            pl.BlockSpec((_BK, DIM), lambda i, j: (j, 0)),
        ],
        out_specs=[
            pl.BlockSpec((_BN, 1), lambda i, j: (i, 0)),
            pl.BlockSpec((_BN, _BK),
                         lambda i, j: (jnp.maximum(i - 1, 0),
                                       jnp.where(i == 0, 0, j))),
            pl.BlockSpec(memory_space=pltpu.SMEM),
        ],
        out_shape=[
            jax.ShapeDtypeStruct((N_TOKENS, 1), jnp.int32),
            jax.ShapeDtypeStruct((N_TOKENS, NUM_CODES), jnp.float32),
            jax.ShapeDtypeStruct((1, 1), jnp.float32),
        ],
        scratch_shapes=[
            pltpu.VMEM((_BN, 1), jnp.float32),
            pltpu.VMEM((8, _BK), jnp.float32),
            pltpu.VMEM((8, _BK), jnp.int32),
            pltpu.VMEM((_BN, 1), jnp.int32),
            pltpu.VMEM((_BN, 1), jnp.float32),
        ],
    )(flat_x, emb)


def _enc_last_body(dummy_ref, idx_ref, enc_ref, coli_ref):
    j = pl.program_id(0)
    del dummy_ref

    @pl.when(j == 0)
    def _():
        coli_ref[...] = lax.broadcasted_iota(jnp.int32, (8, _BK), 1)

    idx_adj = idx_ref[...] - j * _BK
    eq = coli_ref[0:1, :] == idx_adj
    enc_ref[...] = jnp.where(eq, jnp.float32(1.0), jnp.float32(0.0))


def _run_enc_last(enc_part, idx2d):
    return pl.pallas_call(
        _enc_last_body,
        grid=(_KB,),
        in_specs=[
            pl.BlockSpec((8, 128), lambda j: (0, 0)),
            pl.BlockSpec((_BN, 1), lambda j: (_NB - 1, 0)),
        ],
        out_specs=pl.BlockSpec((_BN, _BK), lambda j: (_NB - 1, j)),
        out_shape=jax.ShapeDtypeStruct((N_TOKENS, NUM_CODES), jnp.float32),
        scratch_shapes=[pltpu.VMEM((8, _BK), jnp.int32)],
        input_output_aliases={0: 0},
    )(enc_part, idx2d)


def _make_sc_gather():
    info = plsc.get_sparse_core_info()
    nc, ns = info.num_cores, info.num_subcores
    nw = nc * ns
    b_per_w = N_TOKENS // nw
    ch = 128
    nchunks = b_per_w // ch
    mesh = plsc.VectorSubcoreMesh(core_axis_name="c", subcore_axis_name="s")

    @functools.partial(
        pl.kernel,
        mesh=mesh,
        out_type=jax.ShapeDtypeStruct((N_TOKENS, DIM), jnp.float32),
        scratch_types=[
            pltpu.VMEM((ch,), jnp.int32),
            pltpu.VMEM((ch,), jnp.int32),
            pltpu.VMEM((ch, DIM), jnp.float32),
            pltpu.VMEM((ch, DIM), jnp.float32),
            pltpu.SemaphoreType.DMA,
            pltpu.SemaphoreType.DMA,
        ],
    )
    def sc_gather(table_hbm, idx_hbm, out_hbm,
                  idx_v0, idx_v1, rows_v0, rows_v1, sem0, sem1):
        wid = lax.axis_index("s") * nc + lax.axis_index("c")
        base = wid * b_per_w
        assert nchunks == 2
        pltpu.sync_copy(idx_hbm.at[pl.ds(base, ch)], idx_v0)
        pltpu.sync_copy(idx_hbm.at[pl.ds(base + ch, ch)], idx_v1)
        cp0 = pltpu.async_copy(table_hbm.at[idx_v0], rows_v0, sem0)
        cp1 = pltpu.async_copy(table_hbm.at[idx_v1], rows_v1, sem1)
        cp0.wait()
        st0 = pltpu.async_copy(rows_v0, out_hbm.at[pl.ds(base, ch)], sem0)
        cp1.wait()
        st1 = pltpu.async_copy(rows_v1, out_hbm.at[pl.ds(base + ch, ch)], sem1)
        st0.wait()
        st1.wait()

    return sc_gather


_sc_gather = None


_BN3 = 1024


def _st_body(x_ref, q_ref, st_ref, loss_ref):
    i = pl.program_id(0)
    x = x_ref[...]
    q = q_ref[...]
    d = q - x
    st_ref[...] = x + d
    s = jnp.sum(d * d)

    @pl.when(i == 0)
    def _():
        loss_ref[0, 0] = s

    @pl.when(i != 0)
    def _():
        loss_ref[0, 0] += s


def _run_st(flat_x, q_flat):
    return pl.pallas_call(
        _st_body,
        grid=(N_TOKENS // _BN3,),
        in_specs=[
            pl.BlockSpec((_BN3, DIM), lambda i: (i, 0)),
            pl.BlockSpec((_BN3, DIM), lambda i: (i, 0)),
        ],
        out_specs=[
            pl.BlockSpec((_BN3, DIM), lambda i: (i, 0)),
            pl.BlockSpec(memory_space=pltpu.SMEM),
        ],
        out_shape=[
            jax.ShapeDtypeStruct((N_TOKENS, DIM), jnp.float32),
            jax.ShapeDtypeStruct((1, 1), jnp.float32),
        ],
    )(flat_x, q_flat)


def kernel(inputs, embedding_weight):
    global _sc_gather
    if _sc_gather is None:
        _sc_gather = _make_sc_gather()
    seqlen, bs, d = inputs.shape
    flat = inputs.reshape(-1, d)
    idx2d, enc_part, loss_sum = _run_fused(flat, embedding_weight)
    q_flat = _sc_gather(embedding_weight, idx2d.reshape(-1))
    encodings = _run_enc_last(enc_part, idx2d)
    loss = COMMIT * (loss_sum[0, 0] / jnp.float32(N_TOKENS * DIM))
    return (
        loss,
        q_flat.reshape(seqlen, bs, d),
        encodings.reshape(seqlen, bs, NUM_CODES),
        idx2d,
    )


def _argmin_only_body(x_ref, e_ref, idx_ref, minv_ref, colf_ref):
    i = pl.program_id(0)
    j = pl.program_id(1)

    @pl.when(jnp.logical_and(i == 0, j == 0))
    def _():
        colf_ref[...] = lax.broadcasted_iota(
            jnp.int32, (8, _BK), 1).astype(jnp.float32)

    x = x_ref[...]
    e = e_ref[...]
    en = jnp.sum(e * e, axis=1)
    e2 = -2.0 * e
    mm2 = lax.dot_general(x, e2, (((1,), (1,)), ((), ())),
                          preferred_element_type=jnp.float32)
    xn = jnp.sum(x * x, axis=1, keepdims=True)
    dist = (xn + en[None, :]) + mm2
    m = jnp.min(dist, axis=1, keepdims=True)
    rowf = jnp.min(jnp.where(dist == m, colf_ref[0:1, :], jnp.float32(_BK)),
                   axis=1, keepdims=True)
    lidx = rowf.astype(jnp.int32) + j * _BK

    @pl.when(j == 0)
    def _():
        minv_ref[...] = m
        idx_ref[...] = lidx

    @pl.when(j != 0)
    def _():
        better = m < minv_ref[...]
        idx_ref[...] = jnp.where(better, lidx, idx_ref[...])
        minv_ref[...] = jnp.where(better, m, minv_ref[...])


def _run_argmin_only(flat_x, emb):
    return pl.pallas_call(
        _argmin_only_body,
        grid=(N_TOKENS // _BN, _KB),
        in_specs=[
            pl.BlockSpec((_BN, DIM), lambda i, j: (i, 0)),
            pl.BlockSpec((_BK, DIM), lambda i, j: (j, 0)),
        ],
        out_specs=pl.BlockSpec((_BN, 1), lambda i, j: (i, 0)),
        out_shape=jax.ShapeDtypeStruct((N_TOKENS, 1), jnp.int32),
        scratch_shapes=[pltpu.VMEM((_BN, 1), jnp.float32),
                        pltpu.VMEM((8, _BK), jnp.float32)],
    )(flat_x, emb)


# speedup vs baseline: 1.1281x; 1.1281x over previous
import functools

import jax
import jax.numpy as jnp
from jax import lax
from jax.experimental import pallas as pl
from jax.experimental.pallas import tpu as pltpu
from jax.experimental.pallas import tpu_sc as plsc

NUM_CODES = 8192
DIM = 256
N_TOKENS = 8192
COMMIT = 0.25


_BN = 2048
_BK = 1024
_KB = NUM_CODES // _BK
_BK2 = 2048
_KE = NUM_CODES // _BK2


_NB = N_TOKENS // _BN


def _fused_body(x_ref, e_ref, idx_ref, enc_ref, loss_ref,
                minv_ref, colf_ref, coli_ref, idxp_ref):
    i = pl.program_id(0)
    j = pl.program_id(1)

    @pl.when(jnp.logical_and(i == 0, j == 0))
    def _():
        it = lax.broadcasted_iota(jnp.int32, (8, _BK), 1)
        coli_ref[...] = it
        colf_ref[...] = it.astype(jnp.float32)

    @pl.when(i > 0)
    def _():
        idx_adj = idxp_ref[...] - j * _BK
        eq = coli_ref[0:1, :] == idx_adj
        enc_ref[...] = jnp.where(eq, jnp.float32(1.0), jnp.float32(0.0))

    x = x_ref[...]
    e = e_ref[...]
    en = jnp.sum(e * e, axis=1)
    e2 = -2.0 * e
    mm2 = lax.dot_general(x, e2, (((1,), (1,)), ((), ())),
                          preferred_element_type=jnp.float32)
    xn = jnp.sum(x * x, axis=1, keepdims=True)
    dist = (xn + en[None, :]) + mm2
    m = jnp.min(dist, axis=1, keepdims=True)
    rowf = jnp.min(
        jnp.where(dist == m, colf_ref[0:1, :], jnp.float32(_BK)),
        axis=1, keepdims=True)
    lidx = rowf.astype(jnp.int32) + j * _BK

    @pl.when(j == 0)
    def _():
        minv_ref[...] = m
        idx_ref[...] = lidx

    @pl.when(j != 0)
    def _():
        better = m < minv_ref[...]
        idx_ref[...] = jnp.where(better, lidx, idx_ref[...])
        minv_ref[...] = jnp.where(better, m, minv_ref[...])

    @pl.when(j == _KB - 1)
    def _():
        idxp_ref[...] = idx_ref[...]
        s = jnp.sum(minv_ref[...])

        @pl.when(i == 0)
        def _():
            loss_ref[0, 0] = s

        @pl.when(i != 0)
        def _():
            loss_ref[0, 0] += s


def _run_fused(flat_x, emb):
    return pl.pallas_call(
        _fused_body,
        grid=(_NB, _KB),
        in_specs=[
            pl.BlockSpec((_BN, DIM), lambda i, j: (i, 0)),
            pl.BlockSpec((_BK, DIM), lambda i, j: (j, 0)),
        ],
        out_specs=[
            pl.BlockSpec((_BN, 1), lambda i, j: (i, 0)),
            pl.BlockSpec((_BN, _BK),
                         lambda i, j: (jnp.maximum(i - 1, 0),
                                       jnp.where(i == 0, 0, j))),
            pl.BlockSpec(memory_space=pltpu.SMEM),
        ],
        out_shape=[
            jax.ShapeDtypeStruct((N_TOKENS, 1), jnp.int32),
            jax.ShapeDtypeStruct((N_TOKENS, NUM_CODES), jnp.float32),
            jax.ShapeDtypeStruct((1, 1), jnp.float32),
        ],
        scratch_shapes=[
            pltpu.VMEM((_BN, 1), jnp.float32),
            pltpu.VMEM((8, _BK), jnp.float32),
            pltpu.VMEM((8, _BK), jnp.int32),
            pltpu.VMEM((_BN, 1), jnp.int32),
        ],
    )(flat_x, emb)


def _enc_last_body(dummy_ref, idx_ref, enc_ref, coli_ref):
    j = pl.program_id(0)
    del dummy_ref

    @pl.when(j == 0)
    def _():
        coli_ref[...] = lax.broadcasted_iota(jnp.int32, (8, _BK), 1)

    idx_adj = idx_ref[...] - j * _BK
    eq = coli_ref[0:1, :] == idx_adj
    enc_ref[...] = jnp.where(eq, jnp.float32(1.0), jnp.float32(0.0))


def _run_enc_last(enc_part, idx2d):
    return pl.pallas_call(
        _enc_last_body,
        grid=(_KB,),
        in_specs=[
            pl.BlockSpec((8, 128), lambda j: (0, 0)),
            pl.BlockSpec((_BN, 1), lambda j: (_NB - 1, 0)),
        ],
        out_specs=pl.BlockSpec((_BN, _BK), lambda j: (_NB - 1, j)),
        out_shape=jax.ShapeDtypeStruct((N_TOKENS, NUM_CODES), jnp.float32),
        scratch_shapes=[pltpu.VMEM((8, _BK), jnp.int32)],
        input_output_aliases={0: 0},
    )(enc_part, idx2d)


def _make_sc_gather():
    info = plsc.get_sparse_core_info()
    nc, ns = info.num_cores, info.num_subcores
    nw = nc * ns
    b_per_w = N_TOKENS // nw
    ch = 128
    nchunks = b_per_w // ch
    mesh = plsc.VectorSubcoreMesh(core_axis_name="c", subcore_axis_name="s")

    @functools.partial(
        pl.kernel,
        mesh=mesh,
        out_type=jax.ShapeDtypeStruct((N_TOKENS, DIM), jnp.float32),
        scratch_types=[
            pltpu.VMEM((ch,), jnp.int32),
            pltpu.VMEM((ch,), jnp.int32),
            pltpu.VMEM((ch, DIM), jnp.float32),
            pltpu.VMEM((ch, DIM), jnp.float32),
            pltpu.SemaphoreType.DMA,
            pltpu.SemaphoreType.DMA,
        ],
    )
    def sc_gather(table_hbm, idx_hbm, out_hbm,
                  idx_v0, idx_v1, rows_v0, rows_v1, sem0, sem1):
        wid = lax.axis_index("s") * nc + lax.axis_index("c")
        base = wid * b_per_w
        assert nchunks == 2
        pltpu.sync_copy(idx_hbm.at[pl.ds(base, ch)], idx_v0)
        pltpu.sync_copy(idx_hbm.at[pl.ds(base + ch, ch)], idx_v1)
        cp0 = pltpu.async_copy(table_hbm.at[idx_v0], rows_v0, sem0)
        cp1 = pltpu.async_copy(table_hbm.at[idx_v1], rows_v1, sem1)
        cp0.wait()
        st0 = pltpu.async_copy(rows_v0, out_hbm.at[pl.ds(base, ch)], sem0)
        cp1.wait()
        st1 = pltpu.async_copy(rows_v1, out_hbm.at[pl.ds(base + ch, ch)], sem1)
        st0.wait()
        st1.wait()

    return sc_gather


_sc_gather = None


_BN3 = 1024


def _st_body(x_ref, q_ref, st_ref, loss_ref):
    i = pl.program_id(0)
    x = x_ref[...]
    q = q_ref[...]
    d = q - x
    st_ref[...] = x + d
    s = jnp.sum(d * d)

    @pl.when(i == 0)
    def _():
        loss_ref[0, 0] = s

    @pl.when(i != 0)
    def _():
        loss_ref[0, 0] += s


def _run_st(flat_x, q_flat):
    return pl.pallas_call(
        _st_body,
        grid=(N_TOKENS // _BN3,),
        in_specs=[
            pl.BlockSpec((_BN3, DIM), lambda i: (i, 0)),
            pl.BlockSpec((_BN3, DIM), lambda i: (i, 0)),
        ],
        out_specs=[
            pl.BlockSpec((_BN3, DIM), lambda i: (i, 0)),
            pl.BlockSpec(memory_space=pltpu.SMEM),
        ],
        out_shape=[
            jax.ShapeDtypeStruct((N_TOKENS, DIM), jnp.float32),
            jax.ShapeDtypeStruct((1, 1), jnp.float32),
        ],
    )(flat_x, q_flat)


def kernel(inputs, embedding_weight):
    global _sc_gather
    if _sc_gather is None:
        _sc_gather = _make_sc_gather()
    seqlen, bs, d = inputs.shape
    flat = inputs.reshape(-1, d)
    idx2d, enc_part, loss_sum = _run_fused(flat, embedding_weight)
    q_flat = _sc_gather(embedding_weight, idx2d.reshape(-1))
    encodings = _run_enc_last(enc_part, idx2d)
    loss = COMMIT * (loss_sum[0, 0] / jnp.float32(N_TOKENS * DIM))
    return (
        loss,
        q_flat.reshape(seqlen, bs, d),
        encodings.reshape(seqlen, bs, NUM_CODES),
        idx2d,
    )


def _argmin_only_body(x_ref, e_ref, idx_ref, minv_ref, colf_ref):
    i = pl.program_id(0)
    j = pl.program_id(1)

    @pl.when(jnp.logical_and(i == 0, j == 0))
    def _():
        colf_ref[...] = lax.broadcasted_iota(
            jnp.int32, (8, _BK), 1).astype(jnp.float32)

    x = x_ref[...]
    e = e_ref[...]
    en = jnp.sum(e * e, axis=1)
    e2 = -2.0 * e
    mm2 = lax.dot_general(x, e2, (((1,), (1,)), ((), ())),
                          preferred_element_type=jnp.float32)
    xn = jnp.sum(x * x, axis=1, keepdims=True)
    dist = (xn + en[None, :]) + mm2
    m = jnp.min(dist, axis=1, keepdims=True)
    rowf = jnp.min(jnp.where(dist == m, colf_ref[0:1, :], jnp.float32(_BK)),
                   axis=1, keepdims=True)
    lidx = rowf.astype(jnp.int32) + j * _BK

    @pl.when(j == 0)
    def _():
        minv_ref[...] = m
        idx_ref[...] = lidx

    @pl.when(j != 0)
    def _():
        better = m < minv_ref[...]
        idx_ref[...] = jnp.where(better, lidx, idx_ref[...])
        minv_ref[...] = jnp.where(better, m, minv_ref[...])


def _run_argmin_only(flat_x, emb):
    return pl.pallas_call(
        _argmin_only_body,
        grid=(N_TOKENS // _BN, _KB),
        in_specs=[
            pl.BlockSpec((_BN, DIM), lambda i, j: (i, 0)),
            pl.BlockSpec((_BK, DIM), lambda i, j: (j, 0)),
        ],
        out_specs=pl.BlockSpec((_BN, 1), lambda i, j: (i, 0)),
        out_shape=jax.ShapeDtypeStruct((N_TOKENS, 1), jnp.int32),
        scratch_shapes=[pltpu.VMEM((_BN, 1), jnp.float32),
                        pltpu.VMEM((8, _BK), jnp.float32)],
    )(flat_x, emb)


# speedup vs baseline: 1.1319x; 1.0033x over previous
import functools

import jax
import jax.numpy as jnp
from jax import lax
from jax.experimental import pallas as pl
from jax.experimental.pallas import tpu as pltpu
from jax.experimental.pallas import tpu_sc as plsc

NUM_CODES = 8192
DIM = 256
N_TOKENS = 8192
COMMIT = 0.25


_BN = 2048
_BK = 1024
_KB = NUM_CODES // _BK
_NB = N_TOKENS // _BN


def _fused_body(x_ref, e_ref, idx_ref, enc_ref, loss_ref,
                minv_ref, colf_ref, coli_ref, idxp_ref):
    i = pl.program_id(0)
    j = pl.program_id(1)

    @pl.when(jnp.logical_and(i == 0, j == 0))
    def _():
        it = lax.broadcasted_iota(jnp.int32, (8, _BK), 1)
        coli_ref[...] = it
        colf_ref[...] = it.astype(jnp.float32)

    @pl.when(i > 0)
    def _():
        idx_adj = idxp_ref[...] - j * _BK
        eq = coli_ref[0:1, :] == idx_adj
        enc_ref[...] = jnp.where(eq, jnp.float32(1.0), jnp.float32(0.0))

    x = x_ref[...]
    e = e_ref[...]
    en = jnp.sum(e * e, axis=1)
    e2 = -2.0 * e
    mm2 = lax.dot_general(x, e2, (((1,), (1,)), ((), ())),
                          preferred_element_type=jnp.float32)
    xn = jnp.sum(x * x, axis=1, keepdims=True)
    dist = (xn + en[None, :]) + mm2
    m = jnp.min(dist, axis=1, keepdims=True)
    rowf = jnp.min(
        jnp.where(dist == m, colf_ref[0:1, :], jnp.float32(_BK)),
        axis=1, keepdims=True)
    lidx = rowf.astype(jnp.int32) + j * _BK

    @pl.when(j == 0)
    def _():
        minv_ref[...] = m
        idx_ref[...] = lidx

    @pl.when(j != 0)
    def _():
        better = m < minv_ref[...]
        idx_ref[...] = jnp.where(better, lidx, idx_ref[...])
        minv_ref[...] = jnp.where(better, m, minv_ref[...])

    @pl.when(j == _KB - 1)
    def _():
        idxp_ref[...] = idx_ref[...]
        s = jnp.sum(minv_ref[...])

        @pl.when(i == 0)
        def _():
            loss_ref[0, 0] = s

        @pl.when(i != 0)
        def _():
            loss_ref[0, 0] += s


def _run_fused(flat_x, emb):
    return pl.pallas_call(
        _fused_body,
        grid=(_NB, _KB),
        in_specs=[
            pl.BlockSpec((_BN, DIM), lambda i, j: (i, 0)),
            pl.BlockSpec((_BK, DIM), lambda i, j: (j, 0)),
        ],
        out_specs=[
            pl.BlockSpec((_BN, 1), lambda i, j: (i, 0)),
            pl.BlockSpec((_BN, _BK),
                         lambda i, j: (jnp.maximum(i - 1, 0),
                                       jnp.where(i == 0, 0, j))),
            pl.BlockSpec(memory_space=pltpu.SMEM),
        ],
        out_shape=[
            jax.ShapeDtypeStruct((N_TOKENS, 1), jnp.int32),
            jax.ShapeDtypeStruct((N_TOKENS, NUM_CODES), jnp.float32),
            jax.ShapeDtypeStruct((1, 1), jnp.float32),
        ],
        scratch_shapes=[
            pltpu.VMEM((_BN, 1), jnp.float32),
            pltpu.VMEM((8, _BK), jnp.float32),
            pltpu.VMEM((8, _BK), jnp.int32),
            pltpu.VMEM((_BN, 1), jnp.int32),
        ],
    )(flat_x, emb)


def _enc_last_body(dummy_ref, idx_ref, enc_ref, coli_ref):
    j = pl.program_id(0)
    del dummy_ref

    @pl.when(j == 0)
    def _():
        coli_ref[...] = lax.broadcasted_iota(jnp.int32, (8, _BK), 1)

    idx_adj = idx_ref[...] - j * _BK
    eq = coli_ref[0:1, :] == idx_adj
    enc_ref[...] = jnp.where(eq, jnp.float32(1.0), jnp.float32(0.0))


def _run_enc_last(enc_part, idx2d):
    return pl.pallas_call(
        _enc_last_body,
        grid=(_KB,),
        in_specs=[
            pl.BlockSpec((8, 128), lambda j: (0, 0)),
            pl.BlockSpec((_BN, 1), lambda j: (_NB - 1, 0)),
        ],
        out_specs=pl.BlockSpec((_BN, _BK), lambda j: (_NB - 1, j)),
        out_shape=jax.ShapeDtypeStruct((N_TOKENS, NUM_CODES), jnp.float32),
        scratch_shapes=[pltpu.VMEM((8, _BK), jnp.int32)],
        input_output_aliases={0: 0},
    )(enc_part, idx2d)


def _make_sc_gather():
    info = plsc.get_sparse_core_info()
    nc, ns = info.num_cores, info.num_subcores
    nw = nc * ns
    b_per_w = N_TOKENS // nw
    ch = 128
    nchunks = b_per_w // ch
    mesh = plsc.VectorSubcoreMesh(core_axis_name="c", subcore_axis_name="s")

    @functools.partial(
        pl.kernel,
        mesh=mesh,
        out_type=jax.ShapeDtypeStruct((N_TOKENS, DIM), jnp.float32),
        scratch_types=[
            pltpu.VMEM((ch,), jnp.int32),
            pltpu.VMEM((ch,), jnp.int32),
            pltpu.VMEM((ch, DIM), jnp.float32),
            pltpu.VMEM((ch, DIM), jnp.float32),
            pltpu.SemaphoreType.DMA,
            pltpu.SemaphoreType.DMA,
        ],
    )
    def sc_gather(table_hbm, idx_hbm, out_hbm,
                  idx_v0, idx_v1, rows_v0, rows_v1, sem0, sem1):
        wid = lax.axis_index("s") * nc + lax.axis_index("c")
        base = wid * b_per_w
        assert nchunks == 2
        pltpu.sync_copy(idx_hbm.at[pl.ds(base, ch)], idx_v0)
        pltpu.sync_copy(idx_hbm.at[pl.ds(base + ch, ch)], idx_v1)
        cp0 = pltpu.async_copy(table_hbm.at[idx_v0], rows_v0, sem0)
        cp1 = pltpu.async_copy(table_hbm.at[idx_v1], rows_v1, sem1)
        cp0.wait()
        st0 = pltpu.async_copy(rows_v0, out_hbm.at[pl.ds(base, ch)], sem0)
        cp1.wait()
        st1 = pltpu.async_copy(rows_v1, out_hbm.at[pl.ds(base + ch, ch)], sem1)
        st0.wait()
        st1.wait()

    return sc_gather


_sc_gather = None

def kernel(inputs, embedding_weight):
    global _sc_gather
    if _sc_gather is None:
        _sc_gather = _make_sc_gather()
    seqlen, bs, d = inputs.shape
    flat = inputs.reshape(-1, d)
    idx2d, enc_part, loss_sum = _run_fused(flat, embedding_weight)
    q_flat = _sc_gather(embedding_weight, idx2d.reshape(-1))
    encodings = _run_enc_last(enc_part, idx2d)
    loss = COMMIT * (loss_sum[0, 0] / jnp.float32(N_TOKENS * DIM))
    return (
        loss,
        q_flat.reshape(seqlen, bs, d),
        encodings.reshape(seqlen, bs, NUM_CODES),
        idx2d,
    )
